# baseline (speedup 1.0000x reference)
import functools

import jax
import jax.numpy as jnp
from jax import lax
from jax.experimental import pallas as pl
from jax.experimental.pallas import tpu as pltpu
from jax.experimental.pallas import tpu_sc as plsc

NUM_USERS = 1000000
NUM_ITEMS = 1000000
EMBED_DIM = 64
BATCH = 16384

_info = plsc.get_sparse_core_info()
NC = _info.num_cores
NS = _info.num_subcores
L = _info.num_lanes
NW = NC * NS
BPW = BATCH // NW

PASS_COLS = 256
NPASS = (NUM_USERS + PASS_COLS - 1) // PASS_COLS
LAST_LO = ((NUM_USERS - PASS_COLS + 127) // 128) * 128
LCAP = 768
LVECS = LCAP // L
GCAP = 128
GVECS = GCAP // L
NGRP = 8
MCAP = 32
DUMP = BATCH
FCH = 16

_mesh = plsc.VectorSubcoreMesh(core_axis_name="c", subcore_axis_name="s")
_params = pltpu.CompilerParams(needs_layout_passes=False)


def _helpers():
    lane = lax.iota(jnp.int32, L)
    perms = [lane ^ m for m in (1, 2, 4, 8)]
    rowidx = [lane + L * kk for kk in range(EMBED_DIM // L)]
    jconst = [jnp.full((L,), j, jnp.int32) for j in range(L)]
    dnums = lax.GatherDimensionNumbers(
        offset_dims=(), collapsed_slice_dims=(0,), start_index_map=(0,))

    def shuffle(x, idx):
        return lax.gather(x, idx[:, None], dnums, (1,),
                          mode=lax.GatherScatterMode.PROMISE_IN_BOUNDS)

    def lanesum(s):
        for p in perms:
            s = s + shuffle(s, p)
        return s

    return lane, rowidx, jconst, shuffle, lanesum


def _make_extract():
    @functools.partial(
        pl.kernel,
        mesh=_mesh,
        out_type=[
            jax.ShapeDtypeStruct((BATCH + 1, 128), jnp.float32),
            jax.ShapeDtypeStruct((BATCH + 1, 128), jnp.float32),
        ],
        compiler_params=_params,
        scratch_types=[
            pltpu.VMEM((BATCH,), jnp.int32),
            pltpu.VMEM((LCAP + L,), jnp.int32),
            pltpu.VMEM((LCAP + L,), jnp.int32),
            pltpu.VMEM((LCAP + L,), jnp.int32),
            pltpu.VMEM((LCAP + L,), jnp.int32),
            pltpu.VMEM((NGRP * GCAP + L,), jnp.int32),
            pltpu.VMEM((NGRP * GCAP + L,), jnp.int32),
            pltpu.VMEM((NGRP * GCAP + L,), jnp.int32),
            pltpu.VMEM((NGRP * GCAP + L,), jnp.int32),
            pltpu.VMEM((MCAP + L,), jnp.int32),
            pltpu.VMEM((MCAP + L,), jnp.int32),
            pltpu.VMEM((2, EMBED_DIM, PASS_COLS), jnp.float32),
            pltpu.VMEM((2, EMBED_DIM, PASS_COLS), jnp.float32),
            pltpu.VMEM((4, MCAP, 128), jnp.float32),
            pltpu.VMEM((4, MCAP, 128), jnp.float32),
            pltpu.VMEM((4, MCAP), jnp.int32),
            pltpu.VMEM((4, MCAP), jnp.int32),
            [pltpu.SemaphoreType.DMA] * 2,
            [pltpu.SemaphoreType.DMA] * 2,
            [pltpu.SemaphoreType.DMA] * 4,
            [pltpu.SemaphoreType.DMA] * 4,
        ],
    )
    def k1(ut_hbm, it_hbm, uid_hbm, iid_hbm, ustage, istage,
           ids_v, lidu, lposu, lidi, lposi,
           gidu, gposu, gidi, gposi, mid_v, mpos_v,
           pbu, pbi, rbu, rbi, sxu, sxi,
           sems_pu, sems_pi, sems_su, sems_si):
        lane, rowidx, jconst, shuffle, _ = _helpers()
        wid = lax.axis_index("s") * NC + lax.axis_index("c")
        p0 = (NPASS * wid) // NW
        p1 = (NPASS * (wid + 1)) // NW
        npass = p1 - p0

        def issue_pass(t, slot):
            lo = pl.multiple_of(
                jnp.minimum((p0 + t) * PASS_COLS, LAST_LO), 128)
            pltpu.async_copy(ut_hbm.at[:, pl.ds(lo, PASS_COLS)],
                             pbu.at[slot], sems_pu[slot])
            pltpu.async_copy(it_hbm.at[:, pl.ds(lo, PASS_COLS)],
                             pbi.at[slot], sems_pi[slot])

        issue_pass(0, 0)
        issue_pass(1, 1)

        dumpv = lane * 0 + DUMP
        for s in range(4):
            for kk in range(MCAP // L):
                sxu[s, pl.ds(kk * L, L)] = dumpv
                sxi[s, pl.ds(kk * L, L)] = dumpv
            pltpu.async_copy(rbu.at[s], ustage.at[sxu.at[s]], sems_su[s])
            pltpu.async_copy(rbi.at[s], istage.at[sxi.at[s]], sems_si[s])

        def filt(ids_hbm, lid, lpos):
            pltpu.sync_copy(ids_hbm, ids_v)

            def body(kk, ptr):
                for w in range(FCH):
                    off = kk * (FCH * L) + w * L
                    vec = ids_v[pl.ds(off, L)]
                    pv = vec >> 8
                    m = (pv >= p0) & (pv < p1)
                    pos = lane + off
                    plsc.store_compressed(lid.at[pl.ds(ptr, L)], vec, mask=m)
                    plsc.store_compressed(lpos.at[pl.ds(ptr, L)], pos, mask=m)
                    cnt = plsc.all_reduce_population_count(m)[0]
                    ptr = jnp.minimum(ptr + cnt, LCAP)
                return ptr

            return lax.fori_loop(0, BATCH // (FCH * L), body, 0)

        ucnt = filt(uid_hbm, lidu, lposu)
        icnt = filt(iid_hbm, lidi, lposi)

        def bucket(lid, lpos, lcnt, gid, gpos):
            def body(g, gcnts):
                goff = g * GCAP
                glo = p0 + g * 16
                ptr = 0
                for v in range(LVECS):
                    vec = lid[pl.ds(v * L, L)]
                    pos = lpos[pl.ds(v * L, L)]
                    gv = (vec >> 8) - glo
                    m = (gv >= 0) & (gv < 16) & (v * L + lane < lcnt)
                    plsc.store_compressed(gid.at[pl.ds(goff + ptr, L)],
                                          vec, mask=m)
                    plsc.store_compressed(gpos.at[pl.ds(goff + ptr, L)],
                                          pos, mask=m)
                    cnt = plsc.all_reduce_population_count(m)[0]
                    ptr = jnp.minimum(ptr + cnt, GCAP)
                return jnp.where(lane == g, ptr, gcnts)

            return lax.fori_loop(0, NGRP, body,
                                 jnp.zeros((L,), jnp.int32))

        gcnts_u = bucket(lidu, lposu, ucnt, gidu, gposu)
        gcnts_i = bucket(lidi, lposi, icnt, gidi, gposi)

        def process(t, slot, ssl):
            pltpu.make_async_copy(ut_hbm.at[:, pl.ds(0, PASS_COLS)],
                                  pbu.at[slot], sems_pu[slot]).wait()
            pltpu.make_async_copy(it_hbm.at[:, pl.ds(0, PASS_COLS)],
                                  pbi.at[slot], sems_pi[slot]).wait()
            ps = p0 + t
            lo = jnp.minimum(ps * PASS_COLS, LAST_LO)
            g = t >> 4
            goff = g * GCAP
            gvec = lane * 0 + g

            for gid, gpos, gcnts, pbuf, stage, rb, sx, ssem in (
                    (gidu, gposu, gcnts_u, pbu, ustage, rbu, sxu, sems_su),
                    (gidi, gposi, gcnts_i, pbi, istage, rbi, sxi, sems_si)):
                gcb = shuffle(gcnts, gvec)
                mp = 0
                for v in range(GVECS):
                    vec = gid[pl.ds(goff + v * L, L)]
                    pos = gpos[pl.ds(goff + v * L, L)]
                    m = ((vec >> 8) == ps) & (v * L + lane < gcb)
                    plsc.store_compressed(mid_v.at[pl.ds(mp, L)],
                                          vec, mask=m)
                    plsc.store_compressed(mpos_v.at[pl.ds(mp, L)],
                                          pos, mask=m)
                    cnt = plsc.all_reduce_population_count(m)[0]
                    mp = jnp.minimum(mp + cnt, MCAP)
                mcnt = mp

                pltpu.make_async_copy(stage.at[pl.ds(0, MCAP)],
                                      rb.at[ssl], ssem[ssl]).wait()
                for kk in range(MCAP // L):
                    pvv = mpos_v[pl.ds(kk * L, L)]
                    pvv = jnp.where(kk * L + lane < mcnt, pvv, dumpv)
                    sx[ssl, pl.ds(kk * L, L)] = pvv

                    @pl.when(mcnt > kk * L)
                    def _():
                        mv = mid_v[pl.ds(kk * L, L)]
                        colv = jnp.clip(mv - lo, 0, PASS_COLS - 1)
                        for j in range(L):
                            cj = shuffle(colv, jconst[j])
                            for q in range(EMBED_DIM // L):
                                gth = plsc.load_gather(pbuf.at[slot],
                                                       [rowidx[q], cj])
                                rb[ssl, kk * L + j, pl.ds(q * L, L)] = gth
                pltpu.async_copy(rb.at[ssl], stage.at[sx.at[ssl]],
                                 ssem[ssl])

            issue_pass(t + 2, slot)

        def quad(m, carry):
            process(4 * m, 0, 0)
            process(4 * m + 1, 1, 1)
            process(4 * m + 2, 0, 2)
            process(4 * m + 3, 1, 3)
            return carry

        lax.fori_loop(0, (npass + 3) // 4, quad, 0)

        for slot in range(2):
            pltpu.make_async_copy(ut_hbm.at[:, pl.ds(0, PASS_COLS)],
                                  pbu.at[slot], sems_pu[slot]).wait()
            pltpu.make_async_copy(it_hbm.at[:, pl.ds(0, PASS_COLS)],
                                  pbi.at[slot], sems_pi[slot]).wait()
        for s in range(4):
            pltpu.make_async_copy(ustage.at[pl.ds(0, MCAP)],
                                  rbu.at[s], sems_su[s]).wait()
            pltpu.make_async_copy(istage.at[pl.ds(0, MCAP)],
                                  rbi.at[s], sems_si[s]).wait()

    return k1


def _make_dot():
    CH = 128

    @functools.partial(
        pl.kernel,
        mesh=_mesh,
        out_type=jax.ShapeDtypeStruct((BATCH,), jnp.float32),
        compiler_params=_params,
        scratch_types=[
            pltpu.VMEM((CH, 128), jnp.float32),
            pltpu.VMEM((CH, 128), jnp.float32),
            pltpu.VMEM((BPW,), jnp.float32),
        ],
    )
    def k2(ustage, istage, out_hbm, uch, ich, out_v):
        lane, _, _, _, lanesum = _helpers()
        wid = lax.axis_index("s") * NC + lax.axis_index("c")
        base = wid * BPW

        def chunk(c, carry):
            pltpu.sync_copy(ustage.at[pl.ds(base + c * CH, CH)], uch)
            pltpu.sync_copy(istage.at[pl.ds(base + c * CH, CH)], ich)

            def group(gg, carry2):
                res = jnp.zeros((L,), jnp.float32)
                for r in range(L):
                    row = gg * L + r
                    s = None
                    for q in range(EMBED_DIM // L):
                        uu = uch[row, pl.ds(q * L, L)]
                        ii = ich[row, pl.ds(q * L, L)]
                        s = uu * ii if s is None else s + uu * ii
                    res = jnp.where(lane == r, lanesum(s), res)
                y = 1.0 / (1.0 + jnp.exp(-res))
                out_v[pl.ds(c * CH + gg * L, L)] = y
                return carry2

            lax.fori_loop(0, CH // L, group, 0)
            return carry

        lax.fori_loop(0, BPW // CH, chunk, 0)
        pltpu.sync_copy(out_v, out_hbm.at[pl.ds(base, BPW)])

    return k2


_extract_call = _make_extract()
_dot_call = _make_dot()


def kernel(user_emb, item_emb, user_id, item_id):
    uid = jnp.asarray(user_id, jnp.int32)
    iid = jnp.asarray(item_id, jnp.int32)
    ustage, istage = _extract_call(user_emb.T, item_emb.T, uid, iid)
    return _dot_call(ustage, istage)


# speedup vs baseline: 6.8731x; 6.8731x over previous
import functools

import jax
import jax.numpy as jnp
from jax import lax
from jax.experimental import pallas as pl
from jax.experimental.pallas import tpu as pltpu
from jax.experimental.pallas import tpu_sc as plsc

NUM_USERS = 1000000
NUM_ITEMS = 1000000
EMBED_DIM = 64
BATCH = 16384

_info = plsc.get_sparse_core_info()
NC = _info.num_cores
NS = _info.num_subcores
L = _info.num_lanes
NW = NC * NS
BPW = BATCH // NW

PASS_COLS = 256
NPASS = (NUM_USERS + PASS_COLS - 1) // PASS_COLS
LAST_LO = ((NUM_USERS - PASS_COLS + 127) // 128) * 128
LCAP = 768
LVECS = LCAP // L
GCAP = 128
GVECS = GCAP // L
NGRP = 8
MCAP = 32
DUMP = BATCH
FCH = 16

_mesh = plsc.VectorSubcoreMesh(core_axis_name="c", subcore_axis_name="s")
_params = pltpu.CompilerParams(needs_layout_passes=False)


def _helpers():
    lane = lax.iota(jnp.int32, L)
    perms = [lane ^ m for m in (1, 2, 4, 8)]
    rowidx = [lane + L * kk for kk in range(EMBED_DIM // L)]
    jconst = [jnp.full((L,), j, jnp.int32) for j in range(L)]
    dnums = lax.GatherDimensionNumbers(
        offset_dims=(), collapsed_slice_dims=(0,), start_index_map=(0,))

    def shuffle(x, idx):
        return lax.gather(x, idx[:, None], dnums, (1,),
                          mode=lax.GatherScatterMode.PROMISE_IN_BOUNDS)

    def lanesum(s):
        for p in perms:
            s = s + shuffle(s, p)
        return s

    return lane, rowidx, jconst, shuffle, lanesum


def _make_extract():
    @functools.partial(
        pl.kernel,
        mesh=_mesh,
        out_type=[
            jax.ShapeDtypeStruct((BATCH + NW, 128), jnp.float32),
            jax.ShapeDtypeStruct((BATCH + NW, 128), jnp.float32),
        ],
        compiler_params=_params,
        scratch_types=[
            pltpu.VMEM((BATCH,), jnp.int32),
            pltpu.VMEM((LCAP + L,), jnp.int32),
            pltpu.VMEM((LCAP + L,), jnp.int32),
            pltpu.VMEM((LCAP + L,), jnp.int32),
            pltpu.VMEM((LCAP + L,), jnp.int32),
            pltpu.VMEM((NGRP * GCAP + L,), jnp.int32),
            pltpu.VMEM((NGRP * GCAP + L,), jnp.int32),
            pltpu.VMEM((NGRP * GCAP + L,), jnp.int32),
            pltpu.VMEM((NGRP * GCAP + L,), jnp.int32),
            pltpu.VMEM((MCAP + L,), jnp.int32),
            pltpu.VMEM((MCAP + L,), jnp.int32),
            pltpu.VMEM((2, EMBED_DIM, PASS_COLS), jnp.float32),
            pltpu.VMEM((2, EMBED_DIM, PASS_COLS), jnp.float32),
            pltpu.VMEM((4, MCAP, 128), jnp.float32),
            pltpu.VMEM((4, MCAP, 128), jnp.float32),
            pltpu.VMEM((4, MCAP), jnp.int32),
            pltpu.VMEM((4, MCAP), jnp.int32),
            [pltpu.SemaphoreType.DMA] * 2,
            [pltpu.SemaphoreType.DMA] * 2,
            [pltpu.SemaphoreType.DMA] * 4,
            [pltpu.SemaphoreType.DMA] * 4,
        ],
    )
    def k1(ut_hbm, it_hbm, uid_hbm, iid_hbm, ustage, istage,
           ids_v, lidu, lposu, lidi, lposi,
           gidu, gposu, gidi, gposi, mid_v, mpos_v,
           pbu, pbi, rbu, rbi, sxu, sxi,
           sems_pu, sems_pi, sems_su, sems_si):
        lane, rowidx, jconst, shuffle, _ = _helpers()
        wid = lax.axis_index("s") * NC + lax.axis_index("c")
        p0 = (NPASS * wid) // NW
        p1 = (NPASS * (wid + 1)) // NW
        npass = p1 - p0

        def issue_pass(t, slot):
            lo = pl.multiple_of(
                jnp.minimum((p0 + t) * PASS_COLS, LAST_LO), 128)
            pltpu.async_copy(ut_hbm.at[:, pl.ds(lo, PASS_COLS)],
                             pbu.at[slot], sems_pu[slot])
            pltpu.async_copy(it_hbm.at[:, pl.ds(lo, PASS_COLS)],
                             pbi.at[slot], sems_pi[slot])

        issue_pass(0, 0)
        issue_pass(1, 1)

        dumpv = lane * 0 + BATCH + wid
        for s in range(4):
            for kk in range(MCAP // L):
                sxu[s, pl.ds(kk * L, L)] = dumpv
                sxi[s, pl.ds(kk * L, L)] = dumpv
            pltpu.async_copy(rbu.at[s], ustage.at[sxu.at[s]], sems_su[s])
            pltpu.async_copy(rbi.at[s], istage.at[sxi.at[s]], sems_si[s])

        def filt(ids_hbm, lid, lpos):
            pltpu.sync_copy(ids_hbm, ids_v)

            def body(kk, ptr):
                for w in range(FCH):
                    off = kk * (FCH * L) + w * L
                    vec = ids_v[pl.ds(off, L)]
                    pv = vec >> 8
                    m = (pv >= p0) & (pv < p1)
                    pos = lane + off
                    plsc.store_compressed(lid.at[pl.ds(ptr, L)], vec, mask=m)
                    plsc.store_compressed(lpos.at[pl.ds(ptr, L)], pos, mask=m)
                    cnt = plsc.all_reduce_population_count(m)[0]
                    ptr = jnp.minimum(ptr + cnt, LCAP)
                return ptr

            return lax.fori_loop(0, BATCH // (FCH * L), body, 0)

        ucnt = filt(uid_hbm, lidu, lposu)
        icnt = filt(iid_hbm, lidi, lposi)

        def bucket(lid, lpos, lcnt, gid, gpos):
            def body(g, gcnts):
                goff = g * GCAP
                glo = p0 + g * 16
                ptr = 0
                for v in range(LVECS):
                    vec = lid[pl.ds(v * L, L)]
                    pos = lpos[pl.ds(v * L, L)]
                    gv = (vec >> 8) - glo
                    m = (gv >= 0) & (gv < 16) & (v * L + lane < lcnt)
                    plsc.store_compressed(gid.at[pl.ds(goff + ptr, L)],
                                          vec, mask=m)
                    plsc.store_compressed(gpos.at[pl.ds(goff + ptr, L)],
                                          pos, mask=m)
                    cnt = plsc.all_reduce_population_count(m)[0]
                    ptr = jnp.minimum(ptr + cnt, GCAP)
                return jnp.where(lane == g, ptr, gcnts)

            return lax.fori_loop(0, NGRP, body,
                                 jnp.zeros((L,), jnp.int32))

        gcnts_u = bucket(lidu, lposu, ucnt, gidu, gposu)
        gcnts_i = bucket(lidi, lposi, icnt, gidi, gposi)

        def process(t, slot, ssl):
            pltpu.make_async_copy(ut_hbm.at[:, pl.ds(0, PASS_COLS)],
                                  pbu.at[slot], sems_pu[slot]).wait()
            pltpu.make_async_copy(it_hbm.at[:, pl.ds(0, PASS_COLS)],
                                  pbi.at[slot], sems_pi[slot]).wait()
            ps = p0 + t
            lo = jnp.minimum(ps * PASS_COLS, LAST_LO)
            g = t >> 4
            goff = g * GCAP
            gvec = lane * 0 + g

            for gid, gpos, gcnts, pbuf, stage, rb, sx, ssem in (
                    (gidu, gposu, gcnts_u, pbu, ustage, rbu, sxu, sems_su),
                    (gidi, gposi, gcnts_i, pbi, istage, rbi, sxi, sems_si)):
                gcb = shuffle(gcnts, gvec)
                mp = 0
                for v in range(GVECS):
                    vec = gid[pl.ds(goff + v * L, L)]
                    pos = gpos[pl.ds(goff + v * L, L)]
                    m = ((vec >> 8) == ps) & (v * L + lane < gcb)
                    plsc.store_compressed(mid_v.at[pl.ds(mp, L)],
                                          vec, mask=m)
                    plsc.store_compressed(mpos_v.at[pl.ds(mp, L)],
                                          pos, mask=m)
                    cnt = plsc.all_reduce_population_count(m)[0]
                    mp = jnp.minimum(mp + cnt, MCAP)
                mcnt = mp

                pltpu.make_async_copy(stage.at[pl.ds(0, MCAP)],
                                      rb.at[ssl], ssem[ssl]).wait()
                for kk in range(MCAP // L):
                    pvv = mpos_v[pl.ds(kk * L, L)]
                    pvv = jnp.where(kk * L + lane < mcnt, pvv, dumpv)
                    sx[ssl, pl.ds(kk * L, L)] = pvv

                    @pl.when(mcnt > kk * L)
                    def _():
                        mv = mid_v[pl.ds(kk * L, L)]
                        colv = jnp.clip(mv - lo, 0, PASS_COLS - 1)
                        for j in range(L):
                            cj = shuffle(colv, jconst[j])
                            for q in range(EMBED_DIM // L):
                                gth = plsc.load_gather(pbuf.at[slot],
                                                       [rowidx[q], cj])
                                rb[ssl, kk * L + j, pl.ds(q * L, L)] = gth
                pltpu.async_copy(rb.at[ssl], stage.at[sx.at[ssl]],
                                 ssem[ssl])

            issue_pass(t + 2, slot)

        def quad(m, carry):
            process(4 * m, 0, 0)
            process(4 * m + 1, 1, 1)
            process(4 * m + 2, 0, 2)
            process(4 * m + 3, 1, 3)
            return carry

        lax.fori_loop(0, (npass + 3) // 4, quad, 0)

        for slot in range(2):
            pltpu.make_async_copy(ut_hbm.at[:, pl.ds(0, PASS_COLS)],
                                  pbu.at[slot], sems_pu[slot]).wait()
            pltpu.make_async_copy(it_hbm.at[:, pl.ds(0, PASS_COLS)],
                                  pbi.at[slot], sems_pi[slot]).wait()
        for s in range(4):
            pltpu.make_async_copy(ustage.at[pl.ds(0, MCAP)],
                                  rbu.at[s], sems_su[s]).wait()
            pltpu.make_async_copy(istage.at[pl.ds(0, MCAP)],
                                  rbi.at[s], sems_si[s]).wait()

    return k1


def _make_dot():
    CH = 128

    @functools.partial(
        pl.kernel,
        mesh=_mesh,
        out_type=jax.ShapeDtypeStruct((BATCH,), jnp.float32),
        compiler_params=_params,
        scratch_types=[
            pltpu.VMEM((CH, 128), jnp.float32),
            pltpu.VMEM((CH, 128), jnp.float32),
            pltpu.VMEM((BPW,), jnp.float32),
        ],
    )
    def k2(ustage, istage, out_hbm, uch, ich, out_v):
        lane, _, _, _, lanesum = _helpers()
        wid = lax.axis_index("s") * NC + lax.axis_index("c")
        base = wid * BPW

        def chunk(c, carry):
            pltpu.sync_copy(ustage.at[pl.ds(base + c * CH, CH)], uch)
            pltpu.sync_copy(istage.at[pl.ds(base + c * CH, CH)], ich)

            def group(gg, carry2):
                res = jnp.zeros((L,), jnp.float32)
                for r in range(L):
                    row = gg * L + r
                    s = None
                    for q in range(EMBED_DIM // L):
                        uu = uch[row, pl.ds(q * L, L)]
                        ii = ich[row, pl.ds(q * L, L)]
                        s = uu * ii if s is None else s + uu * ii
                    res = jnp.where(lane == r, lanesum(s), res)
                y = 1.0 / (1.0 + jnp.exp(-res))
                out_v[pl.ds(c * CH + gg * L, L)] = y
                return carry2

            lax.fori_loop(0, CH // L, group, 0)
            return carry

        lax.fori_loop(0, BPW // CH, chunk, 0)
        pltpu.sync_copy(out_v, out_hbm.at[pl.ds(base, BPW)])

    return k2


_extract_call = _make_extract()
_dot_call = _make_dot()


def kernel(user_emb, item_emb, user_id, item_id):
    uid = jnp.asarray(user_id, jnp.int32)
    iid = jnp.asarray(item_id, jnp.int32)
    ustage, istage = _extract_call(user_emb.T, item_emb.T, uid, iid)
    return _dot_call(ustage, istage)


# speedup vs baseline: 14.5983x; 2.1240x over previous
import functools

import jax
import jax.numpy as jnp
from jax import lax
from jax.experimental import pallas as pl
from jax.experimental.pallas import tpu as pltpu
from jax.experimental.pallas import tpu_sc as plsc

NUM_USERS = 1000000
NUM_ITEMS = 1000000
EMBED_DIM = 64
BATCH = 16384

_info = plsc.get_sparse_core_info()
NC = _info.num_cores
NS = _info.num_subcores
L = _info.num_lanes
NW = NC * NS
BPW = BATCH // NW

PASS_COLS = 256
NPASS = (NUM_USERS + PASS_COLS - 1) // PASS_COLS
LAST_LO = ((NUM_USERS - PASS_COLS + 127) // 128) * 128
LCAP = 768
LVECS = LCAP // L
GCAP = 128
GVECS = GCAP // L
NGRP = 8
MCAP = 32
DUMP = BATCH
FCH = 16

_mesh = plsc.VectorSubcoreMesh(core_axis_name="c", subcore_axis_name="s")
_params = pltpu.CompilerParams(needs_layout_passes=False)


def _helpers():
    lane = lax.iota(jnp.int32, L)
    perms = [lane ^ m for m in (1, 2, 4, 8)]
    rowidx = [lane + L * kk for kk in range(EMBED_DIM // L)]
    jconst = [jnp.full((L,), j, jnp.int32) for j in range(L)]
    dnums = lax.GatherDimensionNumbers(
        offset_dims=(), collapsed_slice_dims=(0,), start_index_map=(0,))

    def shuffle(x, idx):
        return lax.gather(x, idx[:, None], dnums, (1,),
                          mode=lax.GatherScatterMode.PROMISE_IN_BOUNDS)

    def lanesum(s):
        for p in perms:
            s = s + shuffle(s, p)
        return s

    return lane, rowidx, jconst, shuffle, lanesum


def _make_extract():
    @functools.partial(
        pl.kernel,
        mesh=_mesh,
        out_type=[
            jax.ShapeDtypeStruct((BATCH + NW * 128, 128), jnp.float32),
            jax.ShapeDtypeStruct((BATCH + NW * 128, 128), jnp.float32),
        ],
        compiler_params=_params,
        scratch_types=[
            pltpu.VMEM((BATCH,), jnp.int32),
            pltpu.VMEM((LCAP + L,), jnp.int32),
            pltpu.VMEM((LCAP + L,), jnp.int32),
            pltpu.VMEM((LCAP + L,), jnp.int32),
            pltpu.VMEM((LCAP + L,), jnp.int32),
            pltpu.VMEM((NGRP * GCAP + L,), jnp.int32),
            pltpu.VMEM((NGRP * GCAP + L,), jnp.int32),
            pltpu.VMEM((NGRP * GCAP + L,), jnp.int32),
            pltpu.VMEM((NGRP * GCAP + L,), jnp.int32),
            pltpu.VMEM((MCAP + L,), jnp.int32),
            pltpu.VMEM((MCAP + L,), jnp.int32),
            pltpu.VMEM((2, EMBED_DIM, PASS_COLS), jnp.float32),
            pltpu.VMEM((2, EMBED_DIM, PASS_COLS), jnp.float32),
            pltpu.VMEM((4, MCAP, 128), jnp.float32),
            pltpu.VMEM((4, MCAP, 128), jnp.float32),
            pltpu.VMEM((4, MCAP), jnp.int32),
            pltpu.VMEM((4, MCAP), jnp.int32),
            [pltpu.SemaphoreType.DMA] * 2,
            [pltpu.SemaphoreType.DMA] * 2,
            [pltpu.SemaphoreType.DMA] * 4,
            [pltpu.SemaphoreType.DMA] * 4,
        ],
    )
    def k1(ut_hbm, it_hbm, uid_hbm, iid_hbm, ustage, istage,
           ids_v, lidu, lposu, lidi, lposi,
           gidu, gposu, gidi, gposi, mid_v, mpos_v,
           pbu, pbi, rbu, rbi, sxu, sxi,
           sems_pu, sems_pi, sems_su, sems_si):
        lane, rowidx, jconst, shuffle, _ = _helpers()
        wid = lax.axis_index("s") * NC + lax.axis_index("c")
        p0 = (NPASS * wid) // NW
        p1 = (NPASS * (wid + 1)) // NW
        npass = p1 - p0

        def issue_pass(t, slot):
            lo = pl.multiple_of(
                jnp.minimum((p0 + t) * PASS_COLS, LAST_LO), 128)
            pltpu.async_copy(ut_hbm.at[:, pl.ds(lo, PASS_COLS)],
                             pbu.at[slot], sems_pu[slot])
            pltpu.async_copy(it_hbm.at[:, pl.ds(lo, PASS_COLS)],
                             pbi.at[slot], sems_pi[slot])

        issue_pass(0, 0)
        issue_pass(1, 1)

        dumpbase = BATCH + wid * 128
        for s in range(4):
            for kk in range(MCAP // L):
                dv = dumpbase + s * MCAP + kk * L + lane
                sxu[s, pl.ds(kk * L, L)] = dv
                sxi[s, pl.ds(kk * L, L)] = dv
            pltpu.async_copy(rbu.at[s], ustage.at[sxu.at[s]], sems_su[s])
            pltpu.async_copy(rbi.at[s], istage.at[sxi.at[s]], sems_si[s])

        def filt(ids_hbm, lid, lpos):
            pltpu.sync_copy(ids_hbm, ids_v)

            def body(kk, ptr):
                for w in range(FCH):
                    off = kk * (FCH * L) + w * L
                    vec = ids_v[pl.ds(off, L)]
                    pv = vec >> 8
                    m = (pv >= p0) & (pv < p1)
                    pos = lane + off
                    plsc.store_compressed(lid.at[pl.ds(ptr, L)], vec, mask=m)
                    plsc.store_compressed(lpos.at[pl.ds(ptr, L)], pos, mask=m)
                    cnt = plsc.all_reduce_population_count(m)[0]
                    ptr = jnp.minimum(ptr + cnt, LCAP)
                return ptr

            return lax.fori_loop(0, BATCH // (FCH * L), body, 0)

        ucnt = filt(uid_hbm, lidu, lposu)
        icnt = filt(iid_hbm, lidi, lposi)

        def bucket(lid, lpos, lcnt, gid, gpos):
            def body(g, gcnts):
                goff = g * GCAP
                glo = p0 + g * 16
                ptr = 0
                for v in range(LVECS):
                    vec = lid[pl.ds(v * L, L)]
                    pos = lpos[pl.ds(v * L, L)]
                    gv = (vec >> 8) - glo
                    m = (gv >= 0) & (gv < 16) & (v * L + lane < lcnt)
                    plsc.store_compressed(gid.at[pl.ds(goff + ptr, L)],
                                          vec, mask=m)
                    plsc.store_compressed(gpos.at[pl.ds(goff + ptr, L)],
                                          pos, mask=m)
                    cnt = plsc.all_reduce_population_count(m)[0]
                    ptr = jnp.minimum(ptr + cnt, GCAP)
                return jnp.where(lane == g, ptr, gcnts)

            return lax.fori_loop(0, NGRP, body,
                                 jnp.zeros((L,), jnp.int32))

        gcnts_u = bucket(lidu, lposu, ucnt, gidu, gposu)
        gcnts_i = bucket(lidi, lposi, icnt, gidi, gposi)

        def process(t, slot, ssl):
            pltpu.make_async_copy(ut_hbm.at[:, pl.ds(0, PASS_COLS)],
                                  pbu.at[slot], sems_pu[slot]).wait()
            pltpu.make_async_copy(it_hbm.at[:, pl.ds(0, PASS_COLS)],
                                  pbi.at[slot], sems_pi[slot]).wait()
            ps = p0 + t
            lo = jnp.minimum(ps * PASS_COLS, LAST_LO)
            g = t >> 4
            goff = g * GCAP
            gvec = lane * 0 + g

            for gid, gpos, gcnts, pbuf, stage, rb, sx, ssem in (
                    (gidu, gposu, gcnts_u, pbu, ustage, rbu, sxu, sems_su),
                    (gidi, gposi, gcnts_i, pbi, istage, rbi, sxi, sems_si)):
                gcb = shuffle(gcnts, gvec)
                mp = 0
                for v in range(GVECS):
                    vec = gid[pl.ds(goff + v * L, L)]
                    pos = gpos[pl.ds(goff + v * L, L)]
                    m = ((vec >> 8) == ps) & (v * L + lane < gcb)
                    plsc.store_compressed(mid_v.at[pl.ds(mp, L)],
                                          vec, mask=m)
                    plsc.store_compressed(mpos_v.at[pl.ds(mp, L)],
                                          pos, mask=m)
                    cnt = plsc.all_reduce_population_count(m)[0]
                    mp = jnp.minimum(mp + cnt, MCAP)
                mcnt = mp

                pltpu.make_async_copy(stage.at[pl.ds(0, MCAP)],
                                      rb.at[ssl], ssem[ssl]).wait()
                for kk in range(MCAP // L):
                    pvv = mpos_v[pl.ds(kk * L, L)]
                    dv = dumpbase + ssl * MCAP + kk * L + lane
                    pvv = jnp.where(kk * L + lane < mcnt, pvv, dv)
                    sx[ssl, pl.ds(kk * L, L)] = pvv

                    @pl.when(mcnt > kk * L)
                    def _():
                        mv = mid_v[pl.ds(kk * L, L)]
                        colv = jnp.clip(mv - lo, 0, PASS_COLS - 1)
                        for j in range(L):
                            cj = shuffle(colv, jconst[j])
                            for q in range(EMBED_DIM // L):
                                gth = plsc.load_gather(pbuf.at[slot],
                                                       [rowidx[q], cj])
                                rb[ssl, kk * L + j, pl.ds(q * L, L)] = gth
                pltpu.async_copy(rb.at[ssl], stage.at[sx.at[ssl]],
                                 ssem[ssl])

            issue_pass(t + 2, slot)

        def quad(m, carry):
            process(4 * m, 0, 0)
            process(4 * m + 1, 1, 1)
            process(4 * m + 2, 0, 2)
            process(4 * m + 3, 1, 3)
            return carry

        lax.fori_loop(0, (npass + 3) // 4, quad, 0)

        for slot in range(2):
            pltpu.make_async_copy(ut_hbm.at[:, pl.ds(0, PASS_COLS)],
                                  pbu.at[slot], sems_pu[slot]).wait()
            pltpu.make_async_copy(it_hbm.at[:, pl.ds(0, PASS_COLS)],
                                  pbi.at[slot], sems_pi[slot]).wait()
        for s in range(4):
            pltpu.make_async_copy(ustage.at[pl.ds(0, MCAP)],
                                  rbu.at[s], sems_su[s]).wait()
            pltpu.make_async_copy(istage.at[pl.ds(0, MCAP)],
                                  rbi.at[s], sems_si[s]).wait()

    return k1


def _make_dot():
    CH = 128

    @functools.partial(
        pl.kernel,
        mesh=_mesh,
        out_type=jax.ShapeDtypeStruct((BATCH,), jnp.float32),
        compiler_params=_params,
        scratch_types=[
            pltpu.VMEM((CH, 128), jnp.float32),
            pltpu.VMEM((CH, 128), jnp.float32),
            pltpu.VMEM((BPW,), jnp.float32),
        ],
    )
    def k2(ustage, istage, out_hbm, uch, ich, out_v):
        lane, _, _, _, lanesum = _helpers()
        wid = lax.axis_index("s") * NC + lax.axis_index("c")
        base = wid * BPW

        def chunk(c, carry):
            pltpu.sync_copy(ustage.at[pl.ds(base + c * CH, CH)], uch)
            pltpu.sync_copy(istage.at[pl.ds(base + c * CH, CH)], ich)

            def group(gg, carry2):
                res = jnp.zeros((L,), jnp.float32)
                for r in range(L):
                    row = gg * L + r
                    s = None
                    for q in range(EMBED_DIM // L):
                        uu = uch[row, pl.ds(q * L, L)]
                        ii = ich[row, pl.ds(q * L, L)]
                        s = uu * ii if s is None else s + uu * ii
                    res = jnp.where(lane == r, lanesum(s), res)
                y = 1.0 / (1.0 + jnp.exp(-res))
                out_v[pl.ds(c * CH + gg * L, L)] = y
                return carry2

            lax.fori_loop(0, CH // L, group, 0)
            return carry

        lax.fori_loop(0, BPW // CH, chunk, 0)
        pltpu.sync_copy(out_v, out_hbm.at[pl.ds(base, BPW)])

    return k2


_extract_call = _make_extract()
_dot_call = _make_dot()


def kernel(user_emb, item_emb, user_id, item_id):
    uid = jnp.asarray(user_id, jnp.int32)
    iid = jnp.asarray(item_id, jnp.int32)
    ustage, istage = _extract_call(user_emb.T, item_emb.T, uid, iid)
    return _dot_call(ustage, istage)
